# baseline (speedup 1.0000x reference)
import functools

import jax
import jax.numpy as jnp
from jax import lax
from jax.experimental import pallas as pl
from jax.experimental.pallas import tpu as pltpu
from jax.experimental.pallas import tpu_sc as plsc

B = 8
C = 256
NUM_ANCHORS = 9
DISC = 28
N_SPARSE = 2048
LEVEL_HW = [(64, 64), (32, 32), (16, 16), (8, 8), (4, 4)]
HWS = [h * w for (h, w) in LEVEL_HW]

NUM_WORKERS = 32
TPW = N_SPARSE // NUM_WORKERS


def _gather_body(f0, f1, f2, f3, f4, sb_hbm, sl_hbm, so_hbm, out_hbm,
                 base_v, hw_v, lvl_v, idx_v, g_v, sem_g):
    feats = [f0, f1, f2, f3, f4]
    wid = lax.axis_index("s") * 2 + lax.axis_index("c")
    base = wid * TPW
    pltpu.sync_copy(sb_hbm.at[pl.ds(base, TPW)], base_v)
    pltpu.sync_copy(sl_hbm.at[pl.ds(base, TPW)], lvl_v)
    pltpu.sync_copy(so_hbm.at[pl.ds(base, TPW)], hw_v)
    lanes = lax.iota(jnp.int32, 16)
    for q in range(TPW // 16):
        sl16 = pl.ds(q * 16, 16)
        bv = base_v[sl16]
        lv = lvl_v[sl16]
        ov = hw_v[sl16]
        hw = jnp.where(
            lv == 0, HWS[0],
            jnp.where(lv == 1, HWS[1],
                      jnp.where(lv == 2, HWS[2],
                                jnp.where(lv == 3, HWS[3], HWS[4]))))
        offv = jnp.bitwise_and(ov, hw - 1)
        basev = bv * (C * hw) + offv
        pos0 = (lanes + q * 16) * C

        def chan(c, carry, hw=hw):
            val, pos = carry
            plsc.store_scatter(idx_v, [pos], val)
            return val + hw, pos + 1
        lax.fori_loop(0, C, chan, (basev, pos0))
    def tok(j, carry):
        qb = jnp.bitwise_and(j, jnp.int32(~15))
        t = jnp.bitwise_and(j, jnp.int32(15))
        lgrp = lvl_v[pl.ds(qb, 16)]
        msk = lanes == t
        for lvl in range(5):
            @pl.when(jnp.any(jnp.logical_and(msk, lgrp == lvl)))
            def _(lvl=lvl):
                for h in range(2):
                    pltpu.async_copy(
                        feats[lvl].at[idx_v.at[pl.ds(j * C + h * 128, 128)]],
                        g_v.at[j, pl.ds(h * 128, 128)], sem_g)
        return carry
    lax.fori_loop(0, TPW, tok, jnp.int32(0))
    pltpu.make_async_copy(
        out_hbm.at[pl.ds(base, TPW), :], g_v, sem_g).wait()
    pltpu.sync_copy(g_v, out_hbm.at[pl.ds(base, TPW), :])


def _gather(f0, f1, f2, f3, f4, sb, sl, so):
    mesh = plsc.VectorSubcoreMesh(core_axis_name="c", subcore_axis_name="s")
    return pl.kernel(
        _gather_body,
        out_type=jax.ShapeDtypeStruct((N_SPARSE, C), jnp.float32),
        mesh=mesh,
        compiler_params=pltpu.CompilerParams(needs_layout_passes=False),
        scratch_types=[
            pltpu.VMEM((TPW,), jnp.int32),
            pltpu.VMEM((TPW,), jnp.int32),
            pltpu.VMEM((TPW,), jnp.int32),
            pltpu.VMEM((TPW * C,), jnp.int32),
            pltpu.VMEM((TPW, C), jnp.float32),
            pltpu.SemaphoreType.DMA,
        ],
    )(f0, f1, f2, f3, f4, sb, sl, so)


TOK_TILE = 256
D2 = DISC * DISC


def _head_body(anch_ref, g_ref, w1_ref, b1_ref, wp_ref, bp_ref, out_ref):
    x = jnp.dot(g_ref[...], w1_ref[...], preferred_element_type=jnp.float32)
    x = jnp.maximum(x + b1_ref[...], 0.0)
    acol = anch_ref[...]
    acc = jnp.dot(
        (acol == lax.broadcasted_iota(jnp.int32, (1, NUM_ANCHORS), 1)
         ).astype(jnp.float32),
        bp_ref[...], preferred_element_type=jnp.float32)
    for a in range(NUM_ANCHORS):
        xm = jnp.where(acol == a, x, 0.0)
        acc = acc + jnp.dot(xm, wp_ref[a],
                            preferred_element_type=jnp.float32)
    out_ref[...] = acc


def _head(anchors, g, W1, b1, Wp, bp):
    grid = (N_SPARSE // TOK_TILE,)
    return pl.pallas_call(
        _head_body,
        grid=grid,
        in_specs=[
            pl.BlockSpec((TOK_TILE, 1), lambda i: (i, 0)),
            pl.BlockSpec((TOK_TILE, C), lambda i: (i, 0)),
            pl.BlockSpec((C, C), lambda i: (0, 0)),
            pl.BlockSpec((1, C), lambda i: (0, 0)),
            pl.BlockSpec((NUM_ANCHORS, C, D2), lambda i: (0, 0, 0)),
            pl.BlockSpec((NUM_ANCHORS, D2), lambda i: (0, 0)),
        ],
        out_specs=pl.BlockSpec((TOK_TILE, D2), lambda i: (i, 0)),
        out_shape=jax.ShapeDtypeStruct((N_SPARSE, D2), jnp.float32),
    )(anchors, g, W1, b1, Wp, bp)


def kernel(feat0, feat1, feat2, feat3, feat4,
           sparse_batch, sparse_layers, sparse_off, sparse_anchor_idx,
           W1, b1, Wp, bp):
    f = [x.reshape(-1) for x in (feat0, feat1, feat2, feat3, feat4)]
    g = _gather(f[0], f[1], f[2], f[3], f[4],
                sparse_batch, sparse_layers, sparse_off)
    out = _head(sparse_anchor_idx.reshape(N_SPARSE, 1), g,
                W1, b1.reshape(1, C), Wp, bp)
    return out.reshape(N_SPARSE, DISC, DISC)
